# baseline (speedup 1.0000x reference)
import functools

import jax
import jax.numpy as jnp
from jax import lax
from jax.experimental import pallas as pl
from jax.experimental.pallas import tpu as pltpu
from jax.experimental.pallas import tpu_sc as plsc

DATASET_SIZE = 1000000
EMBED_DIM = 128
BATCH = 16384

_info = plsc.get_sparse_core_info()
_NC, _NS, _L = _info.num_cores, _info.num_subcores, _info.num_lanes
_NW = _NC * _NS
_B_PER_W = BATCH // _NW
_VECS_PER_ROW = EMBED_DIM // _L


_NCHUNK = 4
_ROWS = _B_PER_W // _NCHUNK
_NBUF = 2


@functools.partial(
    pl.kernel,
    out_type=jax.ShapeDtypeStruct((BATCH, EMBED_DIM), jnp.float32),
    mesh=plsc.VectorSubcoreMesh(core_axis_name="c", subcore_axis_name="s"),
    scratch_types=[
        pltpu.VMEM((_B_PER_W,), jnp.int32),
        pltpu.VMEM((_ROWS, EMBED_DIM), jnp.float32),
        pltpu.VMEM((_ROWS, EMBED_DIM), jnp.float32),
        pltpu.SemaphoreType.DMA,
        pltpu.SemaphoreType.DMA,
        pltpu.SemaphoreType.DMA,
        pltpu.SemaphoreType.DMA,
    ],
)
def _gather_relu(idx_hbm, table_hbm, out_hbm, idx_v, buf0, buf1,
                 gsem0, gsem1, ssem0, ssem1):
    wid = lax.axis_index("s") * _NC + lax.axis_index("c")
    base = wid * _B_PER_W
    bufs = (buf0, buf1)
    gsems = (gsem0, gsem1)
    ssems = (ssem0, ssem1)
    pltpu.sync_copy(idx_hbm.at[pl.ds(base, _B_PER_W)], idx_v)

    def start_gather(c):
        b = c % _NBUF
        return pltpu.async_copy(
            table_hbm.at[idx_v.at[pl.ds(c * _ROWS, _ROWS)]], bufs[b], gsems[b])

    def relu_buf(buf):
        def relu_row(i, carry):
            for j in range(_VECS_PER_ROW):
                sl = pl.ds(j * _L, _L)
                buf[i, sl] = jnp.maximum(buf[i, sl], 0.0)
            return carry
        lax.fori_loop(0, _ROWS, relu_row, 0, unroll=4)

    gathers = [None] * _NCHUNK
    scatters = [None] * _NCHUNK
    gathers[0] = start_gather(0)
    for c in range(_NCHUNK):
        b = c % _NBUF
        gathers[c].wait()
        if c + 1 < _NCHUNK:
            if c + 1 >= _NBUF:
                scatters[c + 1 - _NBUF].wait()
            gathers[c + 1] = start_gather(c + 1)
        relu_buf(bufs[b])
        scatters[c] = pltpu.async_copy(
            bufs[b], out_hbm.at[pl.ds(base + c * _ROWS, _ROWS)], ssems[b])
    for c in range(_NCHUNK - _NBUF, _NCHUNK):
        scatters[c].wait()


def kernel(x, table):
    return _gather_relu(x.astype(jnp.int32), table)


# speedup vs baseline: 1.0028x; 1.0028x over previous
import functools

import jax
import jax.numpy as jnp
from jax import lax
from jax.experimental import pallas as pl
from jax.experimental.pallas import tpu as pltpu
from jax.experimental.pallas import tpu_sc as plsc

DATASET_SIZE = 1000000
EMBED_DIM = 128
BATCH = 16384

_info = plsc.get_sparse_core_info()
_NC, _NS, _L = _info.num_cores, _info.num_subcores, _info.num_lanes
_NW = _NC * _NS
_B_PER_W = BATCH // _NW
_VECS_PER_ROW = EMBED_DIM // _L


_NCHUNK = 4
_ROWS = _B_PER_W // _NCHUNK
_NBUF = 2


@functools.partial(
    pl.kernel,
    out_type=jax.ShapeDtypeStruct((BATCH, EMBED_DIM), jnp.float32),
    mesh=plsc.VectorSubcoreMesh(core_axis_name="c", subcore_axis_name="s"),
    scratch_types=[
        pltpu.VMEM((_B_PER_W,), jnp.int32),
        pltpu.VMEM((_ROWS, EMBED_DIM), jnp.float32),
        pltpu.VMEM((_ROWS, EMBED_DIM), jnp.float32),
        pltpu.SemaphoreType.DMA,
        pltpu.SemaphoreType.DMA,
        pltpu.SemaphoreType.DMA,
        pltpu.SemaphoreType.DMA,
    ],
)
def _gather_relu(idx_hbm, table_hbm, out_hbm, idx_v, buf0, buf1,
                 gsem0, gsem1, ssem0, ssem1):
    wid = lax.axis_index("s") * _NC + lax.axis_index("c")
    base = wid * _B_PER_W
    bufs = (buf0, buf1)
    gsems = (gsem0, gsem1)
    ssems = (ssem0, ssem1)
    pltpu.sync_copy(idx_hbm.at[pl.ds(base, _B_PER_W)], idx_v)

    def start_gather(c):
        b = c % _NBUF
        return pltpu.async_copy(
            table_hbm.at[idx_v.at[pl.ds(c * _ROWS, _ROWS)]], bufs[b], gsems[b])

    def relu_buf(buf):
        @plsc.parallel_loop(0, _ROWS, unroll=4)
        def _relu_row(i):
            for j in range(_VECS_PER_ROW):
                sl = pl.ds(j * _L, _L)
                buf[i, sl] = jnp.maximum(buf[i, sl], 0.0)

    gathers = [None] * _NCHUNK
    scatters = [None] * _NCHUNK
    gathers[0] = start_gather(0)
    for c in range(_NCHUNK):
        b = c % _NBUF
        gathers[c].wait()
        if c + 1 < _NCHUNK:
            if c + 1 >= _NBUF:
                scatters[c + 1 - _NBUF].wait()
            gathers[c + 1] = start_gather(c + 1)
        relu_buf(bufs[b])
        scatters[c] = pltpu.async_copy(
            bufs[b], out_hbm.at[pl.ds(base + c * _ROWS, _ROWS)], ssems[b])
    for c in range(_NCHUNK - _NBUF, _NCHUNK):
        scatters[c].wait()


def kernel(x, table):
    return _gather_relu(x.astype(jnp.int32), table)
